# baseline (speedup 1.0000x reference)
import jax
import jax.numpy as jnp
from jax.experimental import pallas as pl

_N = 4096
_D = 128
_ALPHA = 3.0
_K = 20
_BR = 512
_NBLK = _N // _BR
_W = 128
_NC = _N // _W


def _embed_kernel(e1_ref, e2_ref, t1_ref, t2_ref, m1_ref, m2_ref):
    m1_ref[...] = jnp.tanh(_ALPHA * jnp.dot(e1_ref[...], t1_ref[...]))
    m2_ref[...] = jnp.tanh(_ALPHA * jnp.dot(e2_ref[...], t2_ref[...]))


def _slow_mask(a, col):
    work = a
    for _ in range(_K):
        m = jnp.max(work, axis=1, keepdims=True)
        idx = jnp.min(jnp.where(work == m, col, _N), axis=1, keepdims=True)
        work = jnp.where(col == idx, -1.0, work)
    return work < 0.0


def _topk_kernel(m1blk_ref, m2blk_ref, m1_ref, m2_ref, out_ref):
    a1 = jax.lax.dot_general(m1blk_ref[...], m2_ref[...],
                             (((1,), (1,)), ((), ())))
    a2 = jax.lax.dot_general(m2blk_ref[...], m1_ref[...],
                             (((1,), (1,)), ((), ())))
    a = jnp.maximum(jnp.tanh(_ALPHA * (a1 - a2)), 0.0)

    m = jnp.max(a, axis=1, keepdims=True)
    eq_f = (a == m).astype(jnp.float32)
    bmat = (jax.lax.broadcasted_iota(jnp.int32, (_N, _NC), 0) // _W
            == jax.lax.broadcasted_iota(jnp.int32, (_N, _NC), 1)
            ).astype(jnp.float32)
    cc = jnp.dot(eq_f, bmat)
    e = jnp.sum(cc, axis=1, keepdims=True)
    fast_ok = jnp.all(e >= float(_K))

    @pl.when(fast_ok)
    def _():
        tri_strict = (jax.lax.broadcasted_iota(jnp.int32, (_NC, _NC), 0)
                      < jax.lax.broadcasted_iota(jnp.int32, (_NC, _NC), 1)
                      ).astype(jnp.float32)
        excl = jnp.dot(cc, tri_strict)
        bexcl = jnp.dot(excl, bmat.T)
        tri_inc = (jax.lax.broadcasted_iota(jnp.int32, (_W, _W), 0)
                   <= jax.lax.broadcasted_iota(jnp.int32, (_W, _W), 1)
                   ).astype(jnp.float32)
        parts = [jnp.dot(eq_f[:, c * _W:(c + 1) * _W], tri_inc)
                 for c in range(_NC)]
        intra = jnp.concatenate(parts, axis=1)
        rank = bexcl + intra
        keep = (eq_f > 0.0) & (rank <= float(_K))
        v = m / (float(_K) * m + 1e-6)
        out_ref[...] = jnp.where(keep, v, 0.0)

    @pl.when(jnp.logical_not(fast_ok))
    def _():
        col = jax.lax.broadcasted_iota(jnp.int32, a.shape, 1)
        kept = jnp.where(_slow_mask(a, col), a, 0.0)
        d = jnp.sum(kept, axis=1, keepdims=True) + 1e-6
        out_ref[...] = kept / d


def kernel(e1, e2, theta1, theta2):
    m1, m2 = pl.pallas_call(
        _embed_kernel,
        out_shape=[jax.ShapeDtypeStruct((_N, _D), jnp.float32)] * 2,
    )(e1, e2, theta1, theta2)
    out = pl.pallas_call(
        _topk_kernel,
        grid=(_NBLK,),
        in_specs=[
            pl.BlockSpec((_BR, _D), lambda i: (i, 0)),
            pl.BlockSpec((_BR, _D), lambda i: (i, 0)),
            pl.BlockSpec((_N, _D), lambda i: (0, 0)),
            pl.BlockSpec((_N, _D), lambda i: (0, 0)),
        ],
        out_specs=pl.BlockSpec((_BR, _N), lambda i: (i, 0)),
        out_shape=jax.ShapeDtypeStruct((_N, _N), jnp.float32),
    )(m1, m2, m1, m2)
    return out


# speedup vs baseline: 1.0215x; 1.0215x over previous
import jax
import jax.numpy as jnp
from jax.experimental import pallas as pl

_N = 4096
_D = 128
_ALPHA = 3.0
_K = 20
_BR = 128
_NBLK = _N // _BR
_W = 128
_NC = _N // _W


def _embed_kernel(e1_ref, e2_ref, t1_ref, t2_ref, m1_ref, m2_ref):
    m1_ref[...] = jnp.tanh(_ALPHA * jnp.dot(e1_ref[...], t1_ref[...]))
    m2_ref[...] = jnp.tanh(_ALPHA * jnp.dot(e2_ref[...], t2_ref[...]))


def _slow_mask(a, col):
    work = a
    for _ in range(_K):
        m = jnp.max(work, axis=1, keepdims=True)
        idx = jnp.min(jnp.where(work == m, col, _N), axis=1, keepdims=True)
        work = jnp.where(col == idx, -1.0, work)
    return work < 0.0


def _topk_kernel(m1blk_ref, m2blk_ref, m1_ref, m2_ref, out_ref):
    a1 = jax.lax.dot_general(m1blk_ref[...], m2_ref[...],
                             (((1,), (1,)), ((), ())))
    a2 = jax.lax.dot_general(m2blk_ref[...], m1_ref[...],
                             (((1,), (1,)), ((), ())))
    a = jnp.maximum(jnp.tanh(_ALPHA * (a1 - a2)), 0.0)

    m = jnp.max(a, axis=1, keepdims=True)
    eq_f = (a == m).astype(jnp.float32)
    bmat = (jax.lax.broadcasted_iota(jnp.int32, (_N, _NC), 0) // _W
            == jax.lax.broadcasted_iota(jnp.int32, (_N, _NC), 1)
            ).astype(jnp.float32)
    cc = jnp.dot(eq_f, bmat)
    e = jnp.sum(cc, axis=1, keepdims=True)
    fast_ok = jnp.all(e >= float(_K))

    @pl.when(fast_ok)
    def _():
        tri_strict = (jax.lax.broadcasted_iota(jnp.int32, (_NC, _NC), 0)
                      < jax.lax.broadcasted_iota(jnp.int32, (_NC, _NC), 1)
                      ).astype(jnp.float32)
        excl = jnp.dot(cc, tri_strict)
        bexcl = jnp.dot(excl, bmat.T)
        tri_inc = (jax.lax.broadcasted_iota(jnp.int32, (_W, _W), 0)
                   <= jax.lax.broadcasted_iota(jnp.int32, (_W, _W), 1)
                   ).astype(jnp.float32)
        parts = [jnp.dot(eq_f[:, c * _W:(c + 1) * _W], tri_inc)
                 for c in range(_NC)]
        intra = jnp.concatenate(parts, axis=1)
        rank = bexcl + intra
        keep = (eq_f > 0.0) & (rank <= float(_K))
        v = m / (float(_K) * m + 1e-6)
        out_ref[...] = jnp.where(keep, v, 0.0)

    @pl.when(jnp.logical_not(fast_ok))
    def _():
        col = jax.lax.broadcasted_iota(jnp.int32, a.shape, 1)
        kept = jnp.where(_slow_mask(a, col), a, 0.0)
        d = jnp.sum(kept, axis=1, keepdims=True) + 1e-6
        out_ref[...] = kept / d


def kernel(e1, e2, theta1, theta2):
    m1, m2 = pl.pallas_call(
        _embed_kernel,
        out_shape=[jax.ShapeDtypeStruct((_N, _D), jnp.float32)] * 2,
    )(e1, e2, theta1, theta2)
    out = pl.pallas_call(
        _topk_kernel,
        grid=(_NBLK,),
        in_specs=[
            pl.BlockSpec((_BR, _D), lambda i: (i, 0)),
            pl.BlockSpec((_BR, _D), lambda i: (i, 0)),
            pl.BlockSpec((_N, _D), lambda i: (0, 0)),
            pl.BlockSpec((_N, _D), lambda i: (0, 0)),
        ],
        out_specs=pl.BlockSpec((_BR, _N), lambda i: (i, 0)),
        out_shape=jax.ShapeDtypeStruct((_N, _N), jnp.float32),
    )(m1, m2, m1, m2)
    return out


# speedup vs baseline: 1.0456x; 1.0235x over previous
import jax
import jax.numpy as jnp
from jax.experimental import pallas as pl

_N = 4096
_D = 128
_ALPHA = 3.0
_K = 20
_BR = 256
_NBLK = _N // _BR
_W = 128
_NC = _N // _W


def _embed_kernel(e1_ref, e2_ref, t1_ref, t2_ref, m1_ref, m2_ref):
    m1_ref[...] = jnp.tanh(_ALPHA * jnp.dot(e1_ref[...], t1_ref[...]))
    m2_ref[...] = jnp.tanh(_ALPHA * jnp.dot(e2_ref[...], t2_ref[...]))


def _slow_mask(a, col):
    work = a
    for _ in range(_K):
        m = jnp.max(work, axis=1, keepdims=True)
        idx = jnp.min(jnp.where(work == m, col, _N), axis=1, keepdims=True)
        work = jnp.where(col == idx, -1.0, work)
    return work < 0.0


def _topk_kernel(m1blk_ref, m2blk_ref, m1_ref, m2_ref, out_ref):
    a1 = jax.lax.dot_general(m1blk_ref[...], m2_ref[...],
                             (((1,), (1,)), ((), ())))
    a2 = jax.lax.dot_general(m2blk_ref[...], m1_ref[...],
                             (((1,), (1,)), ((), ())))
    a = jnp.maximum(jnp.tanh(_ALPHA * (a1 - a2)), 0.0)

    m = jnp.max(a, axis=1, keepdims=True)
    eq_f = (a == m).astype(jnp.float32)
    bmat = (jax.lax.broadcasted_iota(jnp.int32, (_N, _NC), 0) // _W
            == jax.lax.broadcasted_iota(jnp.int32, (_N, _NC), 1)
            ).astype(jnp.float32)
    cc = jnp.dot(eq_f, bmat)
    e = jnp.sum(cc, axis=1, keepdims=True)
    fast_ok = jnp.all(e >= float(_K))

    @pl.when(fast_ok)
    def _():
        tri_strict = (jax.lax.broadcasted_iota(jnp.int32, (_NC, _NC), 0)
                      < jax.lax.broadcasted_iota(jnp.int32, (_NC, _NC), 1)
                      ).astype(jnp.float32)
        excl = jnp.dot(cc, tri_strict)
        bexcl = jnp.dot(excl, bmat.T)
        tri_inc = (jax.lax.broadcasted_iota(jnp.int32, (_W, _W), 0)
                   <= jax.lax.broadcasted_iota(jnp.int32, (_W, _W), 1)
                   ).astype(jnp.float32)
        parts = [jnp.dot(eq_f[:, c * _W:(c + 1) * _W], tri_inc)
                 for c in range(_NC)]
        intra = jnp.concatenate(parts, axis=1)
        rank = bexcl + intra
        keep = (eq_f > 0.0) & (rank <= float(_K))
        v = m / (float(_K) * m + 1e-6)
        out_ref[...] = jnp.where(keep, v, 0.0)

    @pl.when(jnp.logical_not(fast_ok))
    def _():
        col = jax.lax.broadcasted_iota(jnp.int32, a.shape, 1)
        kept = jnp.where(_slow_mask(a, col), a, 0.0)
        d = jnp.sum(kept, axis=1, keepdims=True) + 1e-6
        out_ref[...] = kept / d


def kernel(e1, e2, theta1, theta2):
    m1, m2 = pl.pallas_call(
        _embed_kernel,
        out_shape=[jax.ShapeDtypeStruct((_N, _D), jnp.float32)] * 2,
    )(e1, e2, theta1, theta2)
    out = pl.pallas_call(
        _topk_kernel,
        grid=(_NBLK,),
        in_specs=[
            pl.BlockSpec((_BR, _D), lambda i: (i, 0)),
            pl.BlockSpec((_BR, _D), lambda i: (i, 0)),
            pl.BlockSpec((_N, _D), lambda i: (0, 0)),
            pl.BlockSpec((_N, _D), lambda i: (0, 0)),
        ],
        out_specs=pl.BlockSpec((_BR, _N), lambda i: (i, 0)),
        out_shape=jax.ShapeDtypeStruct((_N, _N), jnp.float32),
    )(m1, m2, m1, m2)
    return out
